# Initial kernel scaffold; baseline (speedup 1.0000x reference)
import jax
import jax.numpy as jnp
from jax.experimental import pallas as pl


def kernel(x, weight):
    raise NotImplementedError("write your pallas kernel here")


# speedup vs baseline: 1.0947x; 1.0947x over previous
import functools

import jax
import jax.numpy as jnp
from jax import lax
from jax.experimental import pallas as pl
from jax.experimental.pallas import tpu as pltpu
from jax.experimental.pallas import tpu_sc as plsc

NC, NS = 2, 16
NW = NC * NS
D = 32
B = 16384 * 50
IDX_COLS = 128
ROWS = B // IDX_COLS
ROWS_PER_W = ROWS // NW
CH = 8
CHUNK = CH * IDX_COLS
N_OUTER = ROWS_PER_W // CH


def _emb_body(x_hbm, w_hbm, out_hbm, idx_v, rows_v, sem):
    wid = lax.axis_index("s") * NC + lax.axis_index("c")
    row0 = wid * ROWS_PER_W

    def body(i, carry):
        r0 = row0 + i * CH
        pltpu.sync_copy(x_hbm.at[pl.ds(r0, CH)], idx_v)
        copies = [
            pltpu.async_copy(
                w_hbm.at[idx_v.at[j]],
                rows_v.at[pl.ds(j * IDX_COLS, IDX_COLS)],
                sem,
            )
            for j in range(CH)
        ]
        for c in copies:
            c.wait()
        pltpu.sync_copy(rows_v, out_hbm.at[pl.ds(r0 * IDX_COLS, CHUNK)])
        return carry

    lax.fori_loop(0, N_OUTER, body, 0)


@functools.partial(
    pl.kernel,
    out_type=jax.ShapeDtypeStruct((B, D), jnp.float32),
    mesh=plsc.VectorSubcoreMesh(
        core_axis_name="c", subcore_axis_name="s", num_cores=NC, num_subcores=NS
    ),
    scratch_types=[
        pltpu.VMEM((CH, IDX_COLS), jnp.int32),
        pltpu.VMEM((CHUNK, D), jnp.float32),
        pltpu.SemaphoreType.DMA,
    ],
    compiler_params=pltpu.CompilerParams(use_tc_tiling_on_sc=False),
)
def _emb_lookup(x_hbm, w_hbm, out_hbm, idx_v, rows_v, sem):
    _emb_body(x_hbm, w_hbm, out_hbm, idx_v, rows_v, sem)


def kernel(x, weight):
    xs = x.reshape(ROWS, IDX_COLS).astype(jnp.int32)
    out = _emb_lookup(xs, weight)
    return out.reshape(16384, 50, D)


# speedup vs baseline: 1.8859x; 1.7228x over previous
import functools

import jax
import jax.numpy as jnp
from jax import lax
from jax.experimental import pallas as pl
from jax.experimental.pallas import tpu as pltpu
from jax.experimental.pallas import tpu_sc as plsc

NC, NS = 2, 16
NW = NC * NS
D = 32
NB = 16384
NSQ = 50
CB = 1024
NBC = NB // CB
UNITS = NSQ * NBC
UPW = UNITS // NW
NG = CB // 128


def _emb_body(xt_hbm, w_hbm, out_hbm, idx_v, rows_v, sem):
    wid = lax.axis_index("s") * NC + lax.axis_index("c")

    def body(i, carry):
        u = wid * UPW + i
        s_idx = u // NBC
        b0 = (u % NBC) * CB
        pltpu.sync_copy(xt_hbm.at[s_idx, pl.ds(b0, CB)], idx_v)
        copies = [
            pltpu.async_copy(
                w_hbm.at[idx_v.at[pl.ds(k * 128, 128)]],
                rows_v.at[pl.ds(k * 128, 128)],
                sem,
            )
            for k in range(NG)
        ]
        for c in copies:
            c.wait()
        pltpu.sync_copy(rows_v, out_hbm.at[s_idx, pl.ds(b0, CB)])
        return carry

    lax.fori_loop(0, UPW, body, 0)


@functools.partial(
    pl.kernel,
    out_type=jax.ShapeDtypeStruct((NSQ, NB, D), jnp.float32),
    mesh=plsc.VectorSubcoreMesh(
        core_axis_name="c", subcore_axis_name="s", num_cores=NC, num_subcores=NS
    ),
    scratch_types=[
        pltpu.VMEM((CB,), jnp.int32),
        pltpu.VMEM((CB, D), jnp.float32),
        pltpu.SemaphoreType.DMA,
    ],
    compiler_params=pltpu.CompilerParams(use_tc_tiling_on_sc=False),
)
def _emb_lookup(xt_hbm, w_hbm, out_hbm, idx_v, rows_v, sem):
    _emb_body(xt_hbm, w_hbm, out_hbm, idx_v, rows_v, sem)


def kernel(x, weight):
    xt = x.T.astype(jnp.int32)
    out = _emb_lookup(xt, weight)
    return out.transpose(1, 0, 2)
